# baseline (speedup 1.0000x reference)
import functools

import jax
import jax.numpy as jnp
from jax import lax
from jax.experimental import pallas as pl
from jax.experimental.pallas import tpu as pltpu
from jax.experimental.pallas import tpu_sc as plsc

_NC = 2
_NS = 16
_NW = _NC * _NS


@functools.partial(jax.jit, static_argnames=("b_per_w", "group", "sub", "d"))
def _sc_gather(table, idx, *, b_per_w, group, sub, d):
    batch = idx.shape[0]
    nstep = b_per_w // group
    nsub = group // sub
    mesh = plsc.VectorSubcoreMesh(core_axis_name="c", subcore_axis_name="s")

    @functools.partial(
        pl.kernel,
        out_type=jax.ShapeDtypeStruct((batch, d), jnp.float32),
        mesh=mesh,
        scratch_types=[
            pltpu.VMEM((b_per_w,), jnp.int32),
            pltpu.VMEM((group, d), jnp.float32),
            pltpu.SemaphoreType.DMA,
        ],
    )
    def k(table_hbm, idx_hbm, out_hbm, idx_v, rows_v, sem):
        wid = lax.axis_index("s") * _NC + lax.axis_index("c")
        base = wid * b_per_w
        pltpu.sync_copy(idx_hbm.at[pl.ds(base, b_per_w)], idx_v)

        def body(m, _):
            for s in range(nsub):
                pltpu.async_copy(
                    table_hbm.at[idx_v.at[pl.ds(m * group + s * sub, sub)]],
                    rows_v.at[pl.ds(s * sub, sub)],
                    sem,
                )
            for s in range(nsub):
                pltpu.make_async_copy(
                    table_hbm.at[pl.ds(0, sub)],
                    rows_v.at[pl.ds(s * sub, sub)],
                    sem,
                ).wait()
            plsc.subcore_barrier()
            pltpu.sync_copy(rows_v, out_hbm.at[pl.ds(base + m * group, group)])
            plsc.subcore_barrier()
            return 0

        lax.fori_loop(0, nstep, body, 0)

    return k(table, idx)


def kernel(encoding, available_task):
    _, task_num, d = encoding.shape
    table = encoding.reshape(task_num, d)
    idx = available_task.astype(jnp.int32)
    batch = idx.shape[0]
    out = _sc_gather(table, idx, b_per_w=batch // _NW, group=64, sub=16, d=d)
    return out.reshape(1, batch, d)


# speedup vs baseline: 1.0863x; 1.0863x over previous
import functools

import jax
import jax.numpy as jnp
from jax import lax
from jax.experimental import pallas as pl
from jax.experimental.pallas import tpu as pltpu
from jax.experimental.pallas import tpu_sc as plsc

_NC = 2
_NS = 16
_NW = _NC * _NS


@functools.partial(jax.jit, static_argnames=("b_per_w", "chunk", "nbuf", "d"))
def _sc_gather(table, idx, *, b_per_w, chunk, nbuf, d):
    batch = idx.shape[0]
    nchunk = b_per_w // chunk
    nstep = nchunk // nbuf
    mesh = plsc.VectorSubcoreMesh(core_axis_name="c", subcore_axis_name="s")

    @functools.partial(
        pl.kernel,
        out_type=jax.ShapeDtypeStruct((batch, d), jnp.float32),
        mesh=mesh,
        scratch_types=[
            pltpu.VMEM((b_per_w,), jnp.int32),
            [pltpu.VMEM((chunk, d), jnp.float32)] * nbuf,
            [pltpu.SemaphoreType.DMA] * nbuf,
            [pltpu.SemaphoreType.DMA] * nbuf,
        ],
    )
    def k(table_hbm, idx_hbm, out_hbm, idx_v, bufs, gsems, wsems):
        wid = lax.axis_index("s") * _NC + lax.axis_index("c")
        base = wid * b_per_w
        pltpu.sync_copy(idx_hbm.at[pl.ds(base, b_per_w)], idx_v)

        def gather(c, b):
            pltpu.async_copy(
                table_hbm.at[idx_v.at[pl.ds(c * chunk, chunk)]], bufs[b], gsems[b]
            )

        def gwait(b):
            pltpu.make_async_copy(
                table_hbm.at[pl.ds(0, chunk)], bufs[b], gsems[b]
            ).wait()

        def write(c, b):
            pltpu.async_copy(
                bufs[b], out_hbm.at[pl.ds(base + c * chunk, chunk)], wsems[b]
            )

        def wwait(b):
            pltpu.make_async_copy(
                bufs[b], out_hbm.at[pl.ds(0, chunk)], wsems[b]
            ).wait()

        for b in range(nbuf):
            gather(b, b)

        def body(p, _):
            for b in range(nbuf):
                c = p * nbuf + b
                gwait(b)
                write(c, b)

                @pl.when(p < nstep - 1)
                def _():
                    wwait(b)
                    gather(c + nbuf, b)

            return 0

        lax.fori_loop(0, nstep, body, 0)
        for b in range(nbuf):
            wwait(b)

    return k(table, idx)


def kernel(encoding, available_task):
    _, task_num, d = encoding.shape
    table = encoding.reshape(task_num, d)
    idx = available_task.astype(jnp.int32)
    batch = idx.shape[0]
    out = _sc_gather(table, idx, b_per_w=batch // _NW, chunk=16, nbuf=4, d=d)
    return out.reshape(1, batch, d)


# speedup vs baseline: 1.0908x; 1.0041x over previous
import functools

import jax
import jax.numpy as jnp
from jax import lax
from jax.experimental import pallas as pl
from jax.experimental.pallas import tpu as pltpu
from jax.experimental.pallas import tpu_sc as plsc

_NC = 2
_NS = 16
_NW = _NC * _NS


@functools.partial(jax.jit, static_argnames=("b_per_w", "chunk", "nbuf", "d"))
def _sc_gather(table, idx, *, b_per_w, chunk, nbuf, d):
    batch = idx.shape[0]
    nchunk = b_per_w // chunk
    nstep = nchunk // nbuf
    mesh = plsc.VectorSubcoreMesh(core_axis_name="c", subcore_axis_name="s")

    @functools.partial(
        pl.kernel,
        out_type=jax.ShapeDtypeStruct((batch, d), jnp.float32),
        mesh=mesh,
        scratch_types=[
            pltpu.VMEM((b_per_w,), jnp.int32),
            [pltpu.VMEM((chunk, d), jnp.float32)] * nbuf,
            pltpu.VMEM_SHARED((_NS, 2, chunk, d), jnp.float32),
            [pltpu.SemaphoreType.DMA] * nbuf,
            [pltpu.SemaphoreType.DMA] * 2,
            [pltpu.SemaphoreType.DMA] * 2,
        ],
    )
    def k(table_hbm, idx_hbm, out_hbm, idx_v, bufs, spm, gsems, l1sems, l2sems):
        sid = lax.axis_index("s")
        wid = lax.axis_index("s") * _NC + lax.axis_index("c")
        base = wid * b_per_w
        pltpu.sync_copy(idx_hbm.at[pl.ds(base, b_per_w)], idx_v)

        def gather(c, b):
            pltpu.async_copy(
                table_hbm.at[idx_v.at[pl.ds(c * chunk, chunk)]], bufs[b], gsems[b]
            )

        def gwait(b):
            pltpu.make_async_copy(
                table_hbm.at[pl.ds(0, chunk)], bufs[b], gsems[b]
            ).wait()

        def leg1(b, m):
            pltpu.async_copy(bufs[b], spm.at[sid, m], l1sems[m])

        def l1wait(b, m):
            pltpu.make_async_copy(bufs[b], spm.at[sid, m], l1sems[m]).wait()

        def leg2(c, m):
            pltpu.async_copy(
                spm.at[sid, m], out_hbm.at[pl.ds(base + c * chunk, chunk)], l2sems[m]
            )

        def l2wait(m):
            pltpu.make_async_copy(
                spm.at[sid, m], out_hbm.at[pl.ds(0, chunk)], l2sems[m]
            ).wait()

        for b in range(nbuf):
            gather(b, b)

        def body(p, _):
            for b in range(nbuf):
                c = p * nbuf + b
                m = b % 2
                gwait(b)
                if b >= 2:
                    l2wait(m)
                else:

                    @pl.when(p > 0)
                    def _():
                        l2wait(m)

                leg1(b, m)
                l1wait(b, m)
                leg2(c, m)

                @pl.when(p < nstep - 1)
                def _():
                    gather(c + nbuf, b)

            return 0

        lax.fori_loop(0, nstep, body, 0)
        for m in range(2):
            l2wait(m)

    return k(table, idx)


def kernel(encoding, available_task):
    _, task_num, d = encoding.shape
    table = encoding.reshape(task_num, d)
    idx = available_task.astype(jnp.int32)
    batch = idx.shape[0]
    out = _sc_gather(table, idx, b_per_w=batch // _NW, chunk=16, nbuf=4, d=d)
    return out.reshape(1, batch, d)
